# Initial kernel scaffold; baseline (speedup 1.0000x reference)
import jax
import jax.numpy as jnp
from jax.experimental import pallas as pl


def kernel(frame_idx, pred_idx, frame_weights):
    raise NotImplementedError("write your pallas kernel here")


# speedup vs baseline: 1.6162x; 1.6162x over previous
import functools

import jax
import jax.numpy as jnp
from jax import lax
from jax.experimental import pallas as pl
from jax.experimental.pallas import tpu as pltpu
from jax.experimental.pallas import tpu_sc as plsc

B = 4096
P = 128
NPRED = 20
NC, NS = 2, 16
NW = NC * NS
BPW = B // NW
GROUPS = BPW // 16

_LN2 = 0.6931471805599453
_SQRT2 = 1.4142135623730951


def _ln(x):
    bits = lax.bitcast_convert_type(x, jnp.int32)
    e = (bits >> 23) - 127
    m = lax.bitcast_convert_type((bits & 0x7FFFFF) | 0x3F800000, jnp.float32)
    big = m > _SQRT2
    m = jnp.where(big, m * 0.5, m)
    ef = e.astype(jnp.float32)
    ef = jnp.where(big, ef + 1.0, ef)
    t = (m - 1.0) / (m + 1.0)
    t2 = t * t
    p = jnp.float32(1.0 / 9.0)
    p = 1.0 / 7.0 + t2 * p
    p = 1.0 / 5.0 + t2 * p
    p = 1.0 / 3.0 + t2 * p
    p = 1.0 + t2 * p
    return ef * _LN2 + 2.0 * t * p


def _make_kernel():
    mesh = plsc.VectorSubcoreMesh(core_axis_name="c", subcore_axis_name="s")

    @functools.partial(
        pl.kernel,
        mesh=mesh,
        out_type=jax.ShapeDtypeStruct((B,), jnp.float32),
        scratch_types=[
            pltpu.VMEM((BPW,), jnp.int32),
            pltpu.VMEM((BPW, NPRED), jnp.int32),
            pltpu.VMEM((BPW, P), jnp.float32),
            pltpu.VMEM((BPW,), jnp.float32),
            pltpu.VMEM((BPW,), jnp.float32),
            pltpu.VMEM((BPW,), jnp.float32),
            pltpu.VMEM((BPW,), jnp.float32),
            pltpu.SemaphoreType.DMA,
        ],
    )
    def scorer(frame_hbm, pred_hbm, table_hbm, out_hbm,
               fidx_v, pidx_v, rows_v, mx_v, se_v, s20_v, out_v, sem):
        wid = lax.axis_index("s") * NC + lax.axis_index("c")
        base = wid * BPW

        pltpu.sync_copy(frame_hbm.at[pl.ds(base, BPW)], fidx_v)
        pltpu.sync_copy(pred_hbm.at[pl.ds(base, BPW), :], pidx_v)
        pltpu.async_copy(table_hbm.at[fidx_v], rows_v, sem).wait()

        lane = jnp.arange(16, dtype=jnp.int32)

        def _permute(v, idx):
            return lax.gather(
                v, idx[:, None],
                dimension_numbers=lax.GatherDimensionNumbers(
                    offset_dims=(), collapsed_slice_dims=(0,),
                    start_index_map=(0,)),
                slice_sizes=(1,),
                mode=lax.GatherScatterMode.PROMISE_IN_BOUNDS)

        def _allmax(v):
            for sh in (8, 4, 2, 1):
                v = jnp.maximum(v, _permute(v, lane ^ sh))
            return v

        def _allsum(v):
            for sh in (8, 4, 2, 1):
                v = v + _permute(v, lane ^ sh)
            return v

        def _row_gather(sl, p):
            phi = p >> 4
            plo = p & 15
            out = jnp.zeros((16,), jnp.float32)
            for k in range(8):
                out = jnp.where(phi == k, _permute(sl[k], plo), out)
            return out

        def group_body(g, _):
            r0 = g * 16
            acc_m = jnp.zeros((16,), jnp.float32)
            acc_s = jnp.zeros((16,), jnp.float32)
            acc_g = jnp.zeros((16,), jnp.float32)
            for j in range(16):
                r = r0 + j
                sl = [rows_v[r, pl.ds(16 * k, 16)] for k in range(8)]
                m8 = sl[0]
                for k in range(1, 8):
                    m8 = jnp.maximum(m8, sl[k])
                m = _allmax(m8)
                z = jnp.exp(sl[0] - m)
                for k in range(1, 8):
                    z = z + jnp.exp(sl[k] - m)
                s = _allsum(z)
                p0 = pidx_v[r, pl.ds(0, 16)]
                p1 = pidx_v[r, pl.ds(NPRED - 16, 16)]
                w0 = _row_gather(sl, p0)
                w1 = _row_gather(sl, p1)
                w1 = jnp.where(lane >= 2 * 16 - NPRED, w1, 0.0)
                g20 = _allsum(w0 + w1)
                sel = lane == j
                acc_m = jnp.where(sel, m, acc_m)
                acc_s = jnp.where(sel, s, acc_s)
                acc_g = jnp.where(sel, g20, acc_g)
            mx_v[pl.ds(r0, 16)] = acc_m
            se_v[pl.ds(r0, 16)] = acc_s
            s20_v[pl.ds(r0, 16)] = acc_g
            return 0

        lax.fori_loop(0, GROUPS, group_body, 0)

        for g in range(GROUPS):
            m = mx_v[pl.ds(g * 16, 16)]
            s = se_v[pl.ds(g * 16, 16)]
            g20 = s20_v[pl.ds(g * 16, 16)]
            out_v[pl.ds(g * 16, 16)] = g20 - float(NPRED) * (m + _ln(s))

        pltpu.sync_copy(out_v, out_hbm.at[pl.ds(base, BPW)])

    return scorer


_scorer = _make_kernel()


def kernel(frame_idx, pred_idx, frame_weights):
    return _scorer(frame_idx, pred_idx, frame_weights)
